# Initial kernel scaffold; baseline (speedup 1.0000x reference)
import jax
import jax.numpy as jnp
from jax.experimental import pallas as pl


def kernel(x, edge_index, W_neigh1, W_root1, W_neigh2, W_root2):
    raise NotImplementedError("write your pallas kernel here")


# speedup vs baseline: 6.2561x; 6.2561x over previous
import jax
import jax.numpy as jnp
from jax import lax
from jax.experimental import pallas as pl
from jax.experimental.pallas import tpu as pltpu
from jax.experimental.pallas import tpu_sc as plsc

_N = 10000
_D = 128
_E = 320000
_NC = 2
_NS = 16
_NW = _NC * _NS
_CHUNK = 128
_N_CHUNKS = _E // _CHUNK
_CHUNKS_PER_W = -(-_N_CHUNKS // _NW)
_ROWS_PER_TILE = 632
_N_PAD = _ROWS_PER_TILE * _NS


def _make_spmm(with_deg: bool):
    mesh = plsc.VectorSubcoreMesh(
        core_axis_name="c", subcore_axis_name="s",
        num_cores=_NC, num_subcores=_NS)
    out_type = [jax.ShapeDtypeStruct((_NC, _N_PAD, _D), jnp.float32)]
    if with_deg:
        out_type.append(jax.ShapeDtypeStruct((_NC, _N), jnp.float32))
    scratch = [
        pltpu.VMEM((_CHUNK,), jnp.int32),
        pltpu.VMEM((_CHUNK,), jnp.int32),
        pltpu.VMEM((_CHUNK, _D), jnp.float32),
        pltpu.VMEM((_CHUNK,), jnp.float32),
        pltpu.VMEM_SHARED((_N_PAD, _D), jnp.float32),
        pltpu.VMEM_SHARED((_N,), jnp.float32),
        pltpu.SemaphoreType.DMA,
    ]

    def body(h_hbm, z2d_hbm, z1d_hbm, cols_hbm, rows_hbm, *rest):
        if with_deg:
            (out_acc, out_deg, idx_c, idx_r, buf, ones_v, acc, dacc,
             sem) = rest
        else:
            out_deg = None
            out_acc, idx_c, idx_r, buf, ones_v, acc, dacc, sem = rest
        cid = lax.axis_index("c")
        sid = lax.axis_index("s")
        wid = sid * _NC + cid

        tile_rows = pl.ds(sid * _ROWS_PER_TILE, _ROWS_PER_TILE)
        pltpu.sync_copy(z2d_hbm, acc.at[tile_rows])
        if with_deg:
            @pl.when(sid == 0)
            def _():
                pltpu.sync_copy(z1d_hbm, dacc)
            for i in range(_CHUNK // 16):
                ones_v[pl.ds(i * 16, 16)] = jnp.ones((16,), jnp.float32)
        plsc.subcore_barrier()

        def step(j, carry):
            c_i = j * _NW + wid

            @pl.when(c_i < _N_CHUNKS)
            def _():
                e0 = pl.multiple_of(c_i * _CHUNK, _CHUNK)
                pltpu.sync_copy(cols_hbm.at[pl.ds(e0, _CHUNK)], idx_c)
                pltpu.sync_copy(rows_hbm.at[pl.ds(e0, _CHUNK)], idx_r)
                pltpu.async_copy(h_hbm.at[idx_c], buf, sem).wait()
                pltpu.sync_copy(buf, acc.at[idx_r], add=True)
                if with_deg:
                    pltpu.sync_copy(ones_v, dacc.at[idx_r], add=True)
            return carry

        lax.fori_loop(0, _CHUNKS_PER_W, step, 0)
        plsc.subcore_barrier()

        pltpu.sync_copy(acc.at[tile_rows], out_acc.at[cid, tile_rows])
        if with_deg:
            @pl.when(sid == 0)
            def _():
                pltpu.sync_copy(dacc, out_deg.at[cid])

    return pl.kernel(body, out_type=out_type, mesh=mesh,
                     scratch_types=scratch)


_spmm_deg = _make_spmm(with_deg=True)
_spmm_nodeg = _make_spmm(with_deg=False)


def _tc1_body(acc_ref, deg_ref, x_ref, wn_ref, wr_ref, h_ref, inv_ref):
    deg = jnp.maximum(deg_ref[0] + deg_ref[1], 1.0)
    inv = 1.0 / deg
    agg = (acc_ref[0, :_N] + acc_ref[1, :_N]) * inv
    h = (jnp.dot(agg, wn_ref[...], preferred_element_type=jnp.float32)
         + jnp.dot(x_ref[...], wr_ref[...], preferred_element_type=jnp.float32))
    h_ref[...] = jnp.maximum(h, 0.0)
    inv_ref[...] = inv


def _tc2_body(acc_ref, inv_ref, h_ref, wn_ref, wr_ref, out_ref):
    agg = (acc_ref[0, :_N] + acc_ref[1, :_N]) * inv_ref[...]
    out_ref[...] = (
        jnp.dot(agg, wn_ref[...], preferred_element_type=jnp.float32)
        + jnp.dot(h_ref[...], wr_ref[...], preferred_element_type=jnp.float32))


_tc1 = pl.pallas_call(
    _tc1_body,
    out_shape=[jax.ShapeDtypeStruct((_N, _D), jnp.float32),
               jax.ShapeDtypeStruct((_N, 1), jnp.float32)])

_tc2 = pl.pallas_call(
    _tc2_body,
    out_shape=jax.ShapeDtypeStruct((_N, _D), jnp.float32))


def kernel(x, edge_index, W_neigh1, W_root1, W_neigh2, W_root2):
    rows = edge_index[0].astype(jnp.int32)
    cols = edge_index[1].astype(jnp.int32)
    z2d = jnp.zeros((_ROWS_PER_TILE, _D), jnp.float32)
    z1d = jnp.zeros((_N,), jnp.float32)

    acc1, deg = _spmm_deg(x, z2d, z1d, cols, rows)
    deg = deg.reshape(_NC, _N, 1)
    h, inv = _tc1(acc1, deg, x, W_neigh1.T, W_root1.T)
    (acc2,) = _spmm_nodeg(h, z2d, z1d, cols, rows)
    return _tc2(acc2, inv, h, W_neigh2.T, W_root2.T)
